# baseline (speedup 1.0000x reference)
import functools

import jax
import jax.numpy as jnp
from jax import lax
from jax.experimental import pallas as pl
from jax.experimental.pallas import tpu as pltpu
from jax.experimental.pallas import tpu_sc as plsc

VOCAB = 1000000
DIM = 64
B = 4096
L = 200
N_ROWS = B * L

_info = plsc.get_sparse_core_info()
NC, NS = _info.num_cores, _info.num_subcores
NW = NC * NS
ROWS_PER_W = N_ROWS // NW
CHUNK = 512
N_CHUNKS = ROWS_PER_W // CHUNK


def _make_kernel():
  mesh = plsc.VectorSubcoreMesh(core_axis_name="c", subcore_axis_name="s")

  @functools.partial(
      pl.kernel,
      mesh=mesh,
      out_type=jax.ShapeDtypeStruct((N_ROWS, DIM), jnp.float32),
      scratch_types=[
          pltpu.VMEM((ROWS_PER_W,), jnp.int32),
          pltpu.VMEM((CHUNK, DIM), jnp.float32),
          pltpu.SemaphoreType.DMA,
      ],
      compiler_params=pltpu.CompilerParams(use_tc_tiling_on_sc=False),
  )
  def k(idx_hbm, table_hbm, out_hbm, idx_v, rows_v, gsem):
    wid = lax.axis_index("s") * NC + lax.axis_index("c")
    base = wid * ROWS_PER_W
    pltpu.sync_copy(idx_hbm.at[pl.ds(base, ROWS_PER_W)], idx_v)

    @pl.loop(0, N_CHUNKS)
    def _(i):
      off = i * CHUNK
      pltpu.async_copy(
          table_hbm.at[idx_v.at[pl.ds(off, CHUNK)]], rows_v, gsem
      ).wait()
      pltpu.sync_copy(rows_v, out_hbm.at[pl.ds(base + off, CHUNK)])

  return k


_gather = _make_kernel()


@jax.jit
def kernel(x, table):
  idx = x.reshape(-1).astype(jnp.int32)
  out = _gather(idx, table)
  return out.reshape(B, L, DIM)


# speedup vs baseline: 1.0248x; 1.0248x over previous
import functools

import jax
import jax.numpy as jnp
from jax import lax
from jax.experimental import pallas as pl
from jax.experimental.pallas import tpu as pltpu
from jax.experimental.pallas import tpu_sc as plsc

VOCAB = 1000000
DIM = 64
B = 4096
L = 200
N_ROWS = B * L

_info = plsc.get_sparse_core_info()
NC, NS = _info.num_cores, _info.num_subcores
NW = NC * NS
ROWS_PER_W = N_ROWS // NW
CHUNK = 512
N_CHUNKS = ROWS_PER_W // CHUNK
HALF = N_CHUNKS // 2


def _make_kernel():
  mesh = plsc.VectorSubcoreMesh(core_axis_name="c", subcore_axis_name="s")

  @functools.partial(
      pl.kernel,
      mesh=mesh,
      out_type=jax.ShapeDtypeStruct((N_ROWS, DIM), jnp.float32),
      scratch_types=[
          pltpu.VMEM((ROWS_PER_W,), jnp.int32),
          pltpu.VMEM((CHUNK, DIM), jnp.float32),
          pltpu.VMEM((CHUNK, DIM), jnp.float32),
          pltpu.SemaphoreType.DMA,
          pltpu.SemaphoreType.DMA,
          pltpu.SemaphoreType.DMA,
          pltpu.SemaphoreType.DMA,
      ],
      compiler_params=pltpu.CompilerParams(use_tc_tiling_on_sc=False),
  )
  def k(idx_hbm, table_hbm, out_hbm, idx_v, buf0, buf1, g0, g1, s0, s1):
    wid = lax.axis_index("s") * NC + lax.axis_index("c")
    base = wid * ROWS_PER_W
    pltpu.sync_copy(idx_hbm.at[pl.ds(base, ROWS_PER_W)], idx_v)

    def start_gather(chunk_i, buf, sem):
      pltpu.async_copy(
          table_hbm.at[idx_v.at[pl.ds(chunk_i * CHUNK, CHUNK)]], buf, sem
      )

    def wait_gather(buf, sem):
      pltpu.make_async_copy(table_hbm.at[idx_v.at[pl.ds(0, CHUNK)]], buf,
                            sem).wait()

    def start_store(chunk_i, buf, sem):
      pltpu.async_copy(buf, out_hbm.at[pl.ds(base + chunk_i * CHUNK, CHUNK)],
                       sem)

    def wait_store(buf, sem):
      pltpu.make_async_copy(buf, out_hbm.at[pl.ds(base, CHUNK)], sem).wait()

    start_gather(0, buf0, g0)

    @pl.loop(0, HALF)
    def _(j):
      i0 = 2 * j
      wait_gather(buf0, g0)

      @pl.when(j > 0)
      def _():
        wait_store(buf1, s1)

      start_gather(i0 + 1, buf1, g1)
      start_store(i0, buf0, s0)

      wait_gather(buf1, g1)

      @pl.when(j < HALF - 1)
      def _():
        wait_store(buf0, s0)
        start_gather(i0 + 2, buf0, g0)

      start_store(i0 + 1, buf1, s1)

    wait_store(buf0, s0)
    wait_store(buf1, s1)

  return k


_gather = _make_kernel()


@jax.jit
def kernel(x, table):
  idx = x.reshape(-1).astype(jnp.int32)
  out = _gather(idx, table)
  return out.reshape(B, L, DIM)
